# baseline (speedup 1.0000x reference)
import functools

import jax
import jax.numpy as jnp
from jax import lax
from jax.experimental import pallas as pl
from jax.experimental.pallas import tpu as pltpu
from jax.experimental.pallas import tpu_sc as plsc

_LANES = 16
_K = 128


_BATCH = 8


@functools.lru_cache(maxsize=None)
def _make_scatter(G, N, nblk):
    assert N % 16 == 0 and G % 2 == 0
    assert nblk % 16 == 0
    bpw = nblk // 16
    z_rows = (((N + _LANES) // _LANES) // 8 + 1) * 8
    n_pad = 16 * z_rows
    per_sc = G // 2
    w_rows = (N // 16) // 8 * 8
    w_tail = N - 16 * w_rows
    assert w_tail % 8 == 0

    mesh = plsc.VectorSubcoreMesh(core_axis_name="c", subcore_axis_name="s")

    @functools.partial(
        pl.kernel,
        mesh=mesh,
        out_type=jax.ShapeDtypeStruct((G * N, 128), jnp.float32),
        scratch_types=[
            pltpu.VMEM((_K,), jnp.int32),
            pltpu.VMEM((_K,), jnp.int32),
            pltpu.VMEM((_K,), jnp.int32),
            pltpu.VMEM((_K, 128), jnp.float32),
            pltpu.VMEM((8, 128), jnp.float32),
            pltpu.VMEM_SHARED((n_pad, 128), jnp.float32),
            pltpu.SemaphoreType.DMA,
        ],
    )
    def scatter_kernel(h_hbm, src_hbm, dst_hbm, out_hbm,
                       sidx, didx, gidx, rows, zbuf, acc, sem):
        scid = lax.axis_index("c")
        sid = lax.axis_index("s")

        zv = jnp.zeros((_LANES,), jnp.float32)

        def _zrow(r, carry):
            for j in range(128 // _LANES):
                zbuf[r, pl.ds(j * _LANES, _LANES)] = zv
            return carry

        lax.fori_loop(0, 8, _zrow, 0)

        for p in range(per_sc):
            g = scid * per_sc + p
            goff = g * N

            def _zcopy(i, carry):
                pltpu.sync_copy(
                    zbuf, acc.at[pl.ds(sid * z_rows + i * 8, 8)])
                return carry

            lax.fori_loop(0, z_rows // 8, _zcopy, 0)
            plsc.subcore_barrier()

            def _blk(i, carry):
                b = sid + i * 16
                e0 = b * _K
                pltpu.sync_copy(src_hbm.at[pl.ds(e0, _K)], sidx)
                pltpu.sync_copy(dst_hbm.at[pl.ds(e0, _K)], didx)
                for c in range(_K // _LANES):
                    sl = pl.ds(c * _LANES, _LANES)
                    gidx[sl] = sidx[sl] + goff
                pltpu.async_copy(h_hbm.at[gidx], rows, sem).wait()
                pltpu.sync_copy(rows, acc.at[didx], add=True)
                return carry

            lax.fori_loop(0, bpw, _blk, 0)
            plsc.subcore_barrier()

            r0 = sid * w_rows
            pltpu.sync_copy(acc.at[pl.ds(r0, w_rows)],
                            out_hbm.at[pl.ds(goff + r0, w_rows)])
            if w_tail:
                @pl.when(sid == 0)
                def _():
                    pltpu.sync_copy(
                        acc.at[pl.ds(16 * w_rows, w_tail)],
                        out_hbm.at[pl.ds(goff + 16 * w_rows, w_tail)])
            plsc.subcore_barrier()

    return scatter_kernel


def _segment_sum_cols(h_flat, src, dst, G, N):
    return _make_scatter(G, N, src.shape[0] // _K)(h_flat, src, dst)


def _tc_linear(agg, h, Wr, Wro, br, bn):
    G, N, _ = agg.shape
    H = Wr.shape[2]
    NB = N // bn

    def body(agg_ref, h_ref, wr_ref, wro_ref, br_ref, z_ref, ps_ref, pq_ref):
        z = jnp.broadcast_to(br_ref[...], (bn, H)).astype(jnp.float32)
        for g in range(G):
            z = z + jnp.dot(agg_ref[g], wr_ref[g],
                            preferred_element_type=jnp.float32)
            z = z + jnp.dot(h_ref[g], wro_ref[g],
                            preferred_element_type=jnp.float32)
        z_ref[...] = z
        ps_ref[...] = jnp.sum(z, axis=0).reshape(1, 1, H)
        pq_ref[...] = jnp.sum(z * z, axis=0).reshape(1, 1, H)

    return pl.pallas_call(
        body,
        grid=(NB,),
        in_specs=[
            pl.BlockSpec((G, bn, 128), lambda i: (0, i, 0)),
            pl.BlockSpec((G, bn, 128), lambda i: (0, i, 0)),
            pl.BlockSpec((G, 128, H), lambda i: (0, 0, 0)),
            pl.BlockSpec((G, 128, H), lambda i: (0, 0, 0)),
            pl.BlockSpec((1, H), lambda i: (0, 0)),
        ],
        out_specs=[
            pl.BlockSpec((bn, H), lambda i: (i, 0)),
            pl.BlockSpec((1, 1, H), lambda i: (i, 0, 0)),
            pl.BlockSpec((1, 1, H), lambda i: (i, 0, 0)),
        ],
        out_shape=[
            jax.ShapeDtypeStruct((N, H), jnp.float32),
            jax.ShapeDtypeStruct((NB, 1, H), jnp.float32),
            jax.ShapeDtypeStruct((NB, 1, H), jnp.float32),
        ],
    )(agg, h, Wr, Wro, br.reshape(1, H))


def _tc_bn(z, h, psum, psumsq, gamma, beta, bn, mode):
    N, H = z.shape
    G = H // 128
    NB = N // bn
    gb = jnp.stack([gamma, beta])

    def stats(ps_ref, pq_ref):
        s = jnp.sum(ps_ref[...], axis=(0, 1))
        q = jnp.sum(pq_ref[...], axis=(0, 1))
        mu = s / N
        var = q / N - mu * mu
        return mu, lax.rsqrt(var + 1e-5)

    if mode == "first":
        def body(z_ref, ps_ref, pq_ref, gb_ref, out_ref):
            mu, inv = stats(ps_ref, pq_ref)
            zn = gb_ref[0] * (z_ref[...] - mu) * inv + gb_ref[1]
            r = jnp.maximum(zn, 0.0)
            for g in range(G):
                out_ref[g] = r[:, g * 128:(g + 1) * 128]

        return pl.pallas_call(
            body,
            grid=(NB,),
            in_specs=[
                pl.BlockSpec((bn, H), lambda i: (i, 0)),
                pl.BlockSpec((NB, 1, H), lambda i: (0, 0, 0)),
                pl.BlockSpec((NB, 1, H), lambda i: (0, 0, 0)),
                pl.BlockSpec((2, H), lambda i: (0, 0)),
            ],
            out_specs=pl.BlockSpec((G, bn, 128), lambda i: (0, i, 0)),
            out_shape=jax.ShapeDtypeStruct((G, N, 128), jnp.float32),
        )(z, psum, psumsq, gb)

    if mode == "mid":
        def body(z_ref, h_ref, ps_ref, pq_ref, gb_ref, out_ref):
            mu, inv = stats(ps_ref, pq_ref)
            zn = gb_ref[0] * (z_ref[...] - mu) * inv + gb_ref[1]
            r = jnp.maximum(zn, 0.0)
            for g in range(G):
                out_ref[g] = h_ref[g] + r[:, g * 128:(g + 1) * 128]

        return pl.pallas_call(
            body,
            grid=(NB,),
            in_specs=[
                pl.BlockSpec((bn, H), lambda i: (i, 0)),
                pl.BlockSpec((G, bn, 128), lambda i: (0, i, 0)),
                pl.BlockSpec((NB, 1, H), lambda i: (0, 0, 0)),
                pl.BlockSpec((NB, 1, H), lambda i: (0, 0, 0)),
                pl.BlockSpec((2, H), lambda i: (0, 0)),
            ],
            out_specs=pl.BlockSpec((G, bn, 128), lambda i: (0, i, 0)),
            out_shape=jax.ShapeDtypeStruct((G, N, 128), jnp.float32),
        )(z, h, psum, psumsq, gb)

    def body(z_ref, h_ref, ps_ref, pq_ref, gb_ref, out_ref):
        mu, inv = stats(ps_ref, pq_ref)
        zn = gb_ref[0] * (z_ref[...] - mu) * inv + gb_ref[1]
        r = jnp.maximum(zn, 0.0)
        hfull = jnp.concatenate([h_ref[g] for g in range(G)], axis=1)
        out_ref[...] = hfull + r

    return pl.pallas_call(
        body,
        grid=(NB,),
        in_specs=[
            pl.BlockSpec((bn, H), lambda i: (i, 0)),
            pl.BlockSpec((G, bn, 128), lambda i: (0, i, 0)),
            pl.BlockSpec((NB, 1, H), lambda i: (0, 0, 0)),
            pl.BlockSpec((NB, 1, H), lambda i: (0, 0, 0)),
            pl.BlockSpec((2, H), lambda i: (0, 0)),
        ],
        out_specs=pl.BlockSpec((bn, H), lambda i: (i, 0)),
        out_shape=jax.ShapeDtypeStruct((N, H), jnp.float32),
    )(z, h, psum, psumsq, gb)


def kernel(x, edge_index, W_rel0, b_rel0, W_root0, gamma0, beta0,
           W_rel, b_rel, W_root, gamma, beta):
    N, D = x.shape
    H = W_rel0.shape[1]
    G0 = D // 128
    G = H // 128
    E = edge_index.shape[1]
    bn = 1000

    src = edge_index[0].astype(jnp.int32)
    dst = edge_index[1].astype(jnp.int32)
    nblk_min = (E + _K - 1) // _K
    bpw = ((nblk_min + 15) // 16 + _BATCH - 1) // _BATCH * _BATCH
    e_pad = 16 * bpw * _K
    if e_pad != E:
        pad_pos = jnp.arange(E, e_pad, dtype=jnp.int32)
        src = jnp.concatenate([src, jnp.zeros((e_pad - E,), jnp.int32)])
        dst = jnp.concatenate([dst, N + (pad_pos // _K) % 16])

    x_b = x.reshape(N, G0, 128).transpose(1, 0, 2)

    agg = _segment_sum_cols(x_b.reshape(G0 * N, 128), src, dst, G0, N)
    z, ps, pq = _tc_linear(agg.reshape(G0, N, 128), x_b,
                           W_rel0.reshape(G0, 128, H),
                           W_root0.reshape(G0, 128, H), b_rel0, bn)
    out = _tc_bn(z, None, ps, pq, gamma0, beta0, bn, "first")

    for i in range(4):
        agg = _segment_sum_cols(out.reshape(G * N, 128), src, dst, G, N)
        z, ps, pq = _tc_linear(agg.reshape(G, N, 128), out,
                               W_rel[i].reshape(G, 128, H),
                               W_root[i].reshape(G, 128, H), b_rel[i], bn)
        mode = "last" if i == 3 else "mid"
        out = _tc_bn(z, out, ps, pq, gamma[i], beta[i], bn, mode)

    return out


# speedup vs baseline: 1.4923x; 1.4923x over previous
import functools

import jax
import jax.numpy as jnp
from jax import lax
from jax.experimental import pallas as pl
from jax.experimental.pallas import tpu as pltpu
from jax.experimental.pallas import tpu_sc as plsc

_LANES = 16
_K = 128


_BATCH = 8


@functools.lru_cache(maxsize=None)
def _make_scatter(G, N, nblk):
    assert N % 16 == 0 and G % 2 == 0
    niter = (nblk + 15) // 16
    z_rows = (((N + _LANES) // _LANES) // 8 + 1) * 8
    n_pad = 16 * z_rows
    per_sc = G // 2
    w_rows = (N // 16) // 8 * 8
    w_tail = N - 16 * w_rows
    assert w_tail % 8 == 0

    mesh = plsc.VectorSubcoreMesh(core_axis_name="c", subcore_axis_name="s")

    @functools.partial(
        pl.kernel,
        mesh=mesh,
        out_type=jax.ShapeDtypeStruct((G * N, 128), jnp.float32),
        scratch_types=[
            pltpu.VMEM((_K,), jnp.int32),
            pltpu.VMEM((_K,), jnp.int32),
            pltpu.VMEM((_K,), jnp.int32),
            pltpu.VMEM((_K, 128), jnp.float32),
            pltpu.VMEM((8, 128), jnp.float32),
            pltpu.VMEM_SHARED((n_pad, 128), jnp.float32),
            pltpu.SemaphoreType.DMA,
        ],
    )
    def scatter_kernel(h_hbm, src_hbm, dst_hbm, out_hbm,
                       sidx, didx, gidx, rows, zbuf, acc, sem):
        scid = lax.axis_index("c")
        sid = lax.axis_index("s")

        zv = jnp.zeros((_LANES,), jnp.float32)

        def _zrow(r, carry):
            for j in range(128 // _LANES):
                zbuf[r, pl.ds(j * _LANES, _LANES)] = zv
            return carry

        lax.fori_loop(0, 8, _zrow, 0)

        for p in range(per_sc):
            g = scid * per_sc + p
            goff = g * N

            def _zcopy(i, carry):
                pltpu.sync_copy(
                    zbuf, acc.at[pl.ds(sid * z_rows + i * 8, 8)])
                return carry

            lax.fori_loop(0, z_rows // 8, _zcopy, 0)
            plsc.subcore_barrier()

            def _blk(i, carry):
                b = sid + i * 16

                @pl.when(b < nblk)
                def _():
                    e0 = b * _K
                    pltpu.sync_copy(src_hbm.at[pl.ds(e0, _K)], sidx)
                    pltpu.sync_copy(dst_hbm.at[pl.ds(e0, _K)], didx)
                    for c in range(_K // _LANES):
                        sl = pl.ds(c * _LANES, _LANES)
                        gidx[sl] = sidx[sl] + goff
                    pltpu.async_copy(h_hbm.at[gidx], rows, sem).wait()
                    pltpu.sync_copy(rows, acc.at[didx], add=True)

                return carry

            lax.fori_loop(0, niter, _blk, 0)
            plsc.subcore_barrier()

            r0 = sid * w_rows
            pltpu.sync_copy(acc.at[pl.ds(r0, w_rows)],
                            out_hbm.at[pl.ds(goff + r0, w_rows)])
            if w_tail:
                @pl.when(sid == 0)
                def _():
                    pltpu.sync_copy(
                        acc.at[pl.ds(16 * w_rows, w_tail)],
                        out_hbm.at[pl.ds(goff + 16 * w_rows, w_tail)])
            plsc.subcore_barrier()

    return scatter_kernel


def _segment_sum_cols(h_flat, src, dst, G, N):
    return _make_scatter(G, N, src.shape[0] // _K)(h_flat, src, dst)


def _tc_linear(agg, h, Wr, Wro, br, bn):
    G, N, _ = agg.shape
    H = Wr.shape[2]
    NB = N // bn

    def body(agg_ref, h_ref, wr_ref, wro_ref, br_ref, z_ref, ps_ref, pq_ref):
        z = jnp.broadcast_to(br_ref[...], (bn, H)).astype(jnp.float32)
        for g in range(G):
            z = z + jnp.dot(agg_ref[g], wr_ref[g],
                            preferred_element_type=jnp.float32)
            z = z + jnp.dot(h_ref[g], wro_ref[g],
                            preferred_element_type=jnp.float32)
        z_ref[...] = z
        ps_ref[...] = jnp.sum(z, axis=0).reshape(1, 1, H)
        pq_ref[...] = jnp.sum(z * z, axis=0).reshape(1, 1, H)

    return pl.pallas_call(
        body,
        grid=(NB,),
        in_specs=[
            pl.BlockSpec((G, bn, 128), lambda i: (0, i, 0)),
            pl.BlockSpec((G, bn, 128), lambda i: (0, i, 0)),
            pl.BlockSpec((G, 128, H), lambda i: (0, 0, 0)),
            pl.BlockSpec((G, 128, H), lambda i: (0, 0, 0)),
            pl.BlockSpec((1, H), lambda i: (0, 0)),
        ],
        out_specs=[
            pl.BlockSpec((bn, H), lambda i: (i, 0)),
            pl.BlockSpec((1, 1, H), lambda i: (i, 0, 0)),
            pl.BlockSpec((1, 1, H), lambda i: (i, 0, 0)),
        ],
        out_shape=[
            jax.ShapeDtypeStruct((N, H), jnp.float32),
            jax.ShapeDtypeStruct((NB, 1, H), jnp.float32),
            jax.ShapeDtypeStruct((NB, 1, H), jnp.float32),
        ],
    )(agg, h, Wr, Wro, br.reshape(1, H))


def _tc_bn(z, h, psum, psumsq, gamma, beta, bn, mode):
    N, H = z.shape
    G = H // 128
    NB = N // bn
    gb = jnp.stack([gamma, beta])

    def stats(ps_ref, pq_ref):
        s = jnp.sum(ps_ref[...], axis=(0, 1))
        q = jnp.sum(pq_ref[...], axis=(0, 1))
        mu = s / N
        var = q / N - mu * mu
        return mu, lax.rsqrt(var + 1e-5)

    if mode == "first":
        def body(z_ref, ps_ref, pq_ref, gb_ref, out_ref):
            mu, inv = stats(ps_ref, pq_ref)
            zn = gb_ref[0] * (z_ref[...] - mu) * inv + gb_ref[1]
            r = jnp.maximum(zn, 0.0)
            for g in range(G):
                out_ref[g] = r[:, g * 128:(g + 1) * 128]

        return pl.pallas_call(
            body,
            grid=(NB,),
            in_specs=[
                pl.BlockSpec((bn, H), lambda i: (i, 0)),
                pl.BlockSpec((NB, 1, H), lambda i: (0, 0, 0)),
                pl.BlockSpec((NB, 1, H), lambda i: (0, 0, 0)),
                pl.BlockSpec((2, H), lambda i: (0, 0)),
            ],
            out_specs=pl.BlockSpec((G, bn, 128), lambda i: (0, i, 0)),
            out_shape=jax.ShapeDtypeStruct((G, N, 128), jnp.float32),
        )(z, psum, psumsq, gb)

    if mode == "mid":
        def body(z_ref, h_ref, ps_ref, pq_ref, gb_ref, out_ref):
            mu, inv = stats(ps_ref, pq_ref)
            zn = gb_ref[0] * (z_ref[...] - mu) * inv + gb_ref[1]
            r = jnp.maximum(zn, 0.0)
            for g in range(G):
                out_ref[g] = h_ref[g] + r[:, g * 128:(g + 1) * 128]

        return pl.pallas_call(
            body,
            grid=(NB,),
            in_specs=[
                pl.BlockSpec((bn, H), lambda i: (i, 0)),
                pl.BlockSpec((G, bn, 128), lambda i: (0, i, 0)),
                pl.BlockSpec((NB, 1, H), lambda i: (0, 0, 0)),
                pl.BlockSpec((NB, 1, H), lambda i: (0, 0, 0)),
                pl.BlockSpec((2, H), lambda i: (0, 0)),
            ],
            out_specs=pl.BlockSpec((G, bn, 128), lambda i: (0, i, 0)),
            out_shape=jax.ShapeDtypeStruct((G, N, 128), jnp.float32),
        )(z, h, psum, psumsq, gb)

    def body(z_ref, h_ref, ps_ref, pq_ref, gb_ref, out_ref):
        mu, inv = stats(ps_ref, pq_ref)
        zn = gb_ref[0] * (z_ref[...] - mu) * inv + gb_ref[1]
        r = jnp.maximum(zn, 0.0)
        hfull = jnp.concatenate([h_ref[g] for g in range(G)], axis=1)
        out_ref[...] = hfull + r

    return pl.pallas_call(
        body,
        grid=(NB,),
        in_specs=[
            pl.BlockSpec((bn, H), lambda i: (i, 0)),
            pl.BlockSpec((G, bn, 128), lambda i: (0, i, 0)),
            pl.BlockSpec((NB, 1, H), lambda i: (0, 0, 0)),
            pl.BlockSpec((NB, 1, H), lambda i: (0, 0, 0)),
            pl.BlockSpec((2, H), lambda i: (0, 0)),
        ],
        out_specs=pl.BlockSpec((bn, H), lambda i: (i, 0)),
        out_shape=jax.ShapeDtypeStruct((N, H), jnp.float32),
    )(z, h, psum, psumsq, gb)


def kernel(x, edge_index, W_rel0, b_rel0, W_root0, gamma0, beta0,
           W_rel, b_rel, W_root, gamma, beta):
    N, D = x.shape
    H = W_rel0.shape[1]
    G0 = D // 128
    G = H // 128
    E = edge_index.shape[1]
    bn = 1000

    src = edge_index[0].astype(jnp.int32)
    dst = edge_index[1].astype(jnp.int32)
    e_pad = ((E + _K - 1) // _K) * _K
    if e_pad != E:
        pad_pos = jnp.arange(E, e_pad, dtype=jnp.int32)
        src = jnp.concatenate([src, jnp.zeros((e_pad - E,), jnp.int32)])
        dst = jnp.concatenate([dst, N + pad_pos % 16])

    x_b = x.reshape(N, G0, 128).transpose(1, 0, 2)

    agg = _segment_sum_cols(x_b.reshape(G0 * N, 128), src, dst, G0, N)
    z, ps, pq = _tc_linear(agg.reshape(G0, N, 128), x_b,
                           W_rel0.reshape(G0, 128, H),
                           W_root0.reshape(G0, 128, H), b_rel0, bn)
    out = _tc_bn(z, None, ps, pq, gamma0, beta0, bn, "first")

    for i in range(4):
        agg = _segment_sum_cols(out.reshape(G * N, 128), src, dst, G, N)
        z, ps, pq = _tc_linear(agg.reshape(G, N, 128), out,
                               W_rel[i].reshape(G, 128, H),
                               W_root[i].reshape(G, 128, H), b_rel[i], bn)
        mode = "last" if i == 3 else "mid"
        out = _tc_bn(z, out, ps, pq, gamma[i], beta[i], bn, mode)

    return out


# speedup vs baseline: 1.5952x; 1.0690x over previous
import functools

import jax
import jax.numpy as jnp
from jax import lax
from jax.experimental import pallas as pl
from jax.experimental.pallas import tpu as pltpu
from jax.experimental.pallas import tpu_sc as plsc

_LANES = 16
_K = 128


_BATCH = 8


@functools.lru_cache(maxsize=None)
def _make_scatter(G, N, nblk):
    assert N % 16 == 0 and G % 2 == 0
    niter = (nblk + 15) // 16
    z_rows = (((N + _LANES) // _LANES) // 8 + 1) * 8
    n_pad = 16 * z_rows
    per_sc = G // 2
    w_rows = (N // 16) // 8 * 8
    w_tail = N - 16 * w_rows
    assert w_tail % 8 == 0

    mesh = plsc.VectorSubcoreMesh(core_axis_name="c", subcore_axis_name="s")

    @functools.partial(
        pl.kernel,
        mesh=mesh,
        out_type=jax.ShapeDtypeStruct((G * N, 128), jnp.float32),
        scratch_types=[
            pltpu.VMEM((2, _K), jnp.int32),
            pltpu.VMEM((2, _K), jnp.int32),
            pltpu.VMEM((2, _K), jnp.int32),
            pltpu.VMEM((2, _K, 128), jnp.float32),
            pltpu.VMEM((8, 128), jnp.float32),
            pltpu.VMEM_SHARED((n_pad, 128), jnp.float32),
            pltpu.SemaphoreType.DMA((2,)),
            pltpu.SemaphoreType.DMA((2,)),
        ],
    )
    def scatter_kernel(h_hbm, src_hbm, dst_hbm, out_hbm,
                       sidx, didx, gidx, rows, zbuf, acc, gsem, ssem):
        scid = lax.axis_index("c")
        sid = lax.axis_index("s")

        zv = jnp.zeros((_LANES,), jnp.float32)

        def _zrow(r, carry):
            for j in range(128 // _LANES):
                zbuf[r, pl.ds(j * _LANES, _LANES)] = zv
            return carry

        lax.fori_loop(0, 8, _zrow, 0)

        for p in range(per_sc):
            g = scid * per_sc + p
            goff = g * N

            def _zcopy(i, carry):
                pltpu.sync_copy(
                    zbuf, acc.at[pl.ds(sid * z_rows + i * 8, 8)])
                return carry

            lax.fori_loop(0, z_rows // 8, _zcopy, 0)
            plsc.subcore_barrier()

            @plsc.parallel_loop(0, niter, 1, unroll=2)
            def _blk(i):
                b = sid + i * 16
                k = i % 2

                @pl.when(b < nblk)
                def _():
                    e0 = b * _K
                    pltpu.async_copy(
                        src_hbm.at[pl.ds(e0, _K)], sidx.at[k], gsem.at[k]
                    ).wait()
                    pltpu.async_copy(
                        dst_hbm.at[pl.ds(e0, _K)], didx.at[k], gsem.at[k]
                    ).wait()
                    for c in range(_K // _LANES):
                        sl = pl.ds(c * _LANES, _LANES)
                        gidx[k, sl] = sidx[k, sl] + goff
                    pltpu.async_copy(
                        h_hbm.at[gidx.at[k]], rows.at[k], gsem.at[k]
                    ).wait()
                    pltpu.async_copy(
                        rows.at[k], acc.at[didx.at[k]], ssem.at[k], add=True
                    ).wait()
            plsc.subcore_barrier()

            r0 = sid * w_rows
            pltpu.sync_copy(acc.at[pl.ds(r0, w_rows)],
                            out_hbm.at[pl.ds(goff + r0, w_rows)])
            if w_tail:
                @pl.when(sid == 0)
                def _():
                    pltpu.sync_copy(
                        acc.at[pl.ds(16 * w_rows, w_tail)],
                        out_hbm.at[pl.ds(goff + 16 * w_rows, w_tail)])
            plsc.subcore_barrier()

    return scatter_kernel


def _segment_sum_cols(h_flat, src, dst, G, N):
    return _make_scatter(G, N, src.shape[0] // _K)(h_flat, src, dst)


def _tc_linear(agg, h, Wr, Wro, br, bn):
    G, N, _ = agg.shape
    H = Wr.shape[2]
    NB = N // bn

    def body(agg_ref, h_ref, wr_ref, wro_ref, br_ref, z_ref, ps_ref, pq_ref):
        z = jnp.broadcast_to(br_ref[...], (bn, H)).astype(jnp.float32)
        for g in range(G):
            z = z + jnp.dot(agg_ref[g], wr_ref[g],
                            preferred_element_type=jnp.float32)
            z = z + jnp.dot(h_ref[g], wro_ref[g],
                            preferred_element_type=jnp.float32)
        z_ref[...] = z
        ps_ref[...] = jnp.sum(z, axis=0).reshape(1, 1, H)
        pq_ref[...] = jnp.sum(z * z, axis=0).reshape(1, 1, H)

    return pl.pallas_call(
        body,
        grid=(NB,),
        in_specs=[
            pl.BlockSpec((G, bn, 128), lambda i: (0, i, 0)),
            pl.BlockSpec((G, bn, 128), lambda i: (0, i, 0)),
            pl.BlockSpec((G, 128, H), lambda i: (0, 0, 0)),
            pl.BlockSpec((G, 128, H), lambda i: (0, 0, 0)),
            pl.BlockSpec((1, H), lambda i: (0, 0)),
        ],
        out_specs=[
            pl.BlockSpec((bn, H), lambda i: (i, 0)),
            pl.BlockSpec((1, 1, H), lambda i: (i, 0, 0)),
            pl.BlockSpec((1, 1, H), lambda i: (i, 0, 0)),
        ],
        out_shape=[
            jax.ShapeDtypeStruct((N, H), jnp.float32),
            jax.ShapeDtypeStruct((NB, 1, H), jnp.float32),
            jax.ShapeDtypeStruct((NB, 1, H), jnp.float32),
        ],
    )(agg, h, Wr, Wro, br.reshape(1, H))


def _tc_bn(z, h, psum, psumsq, gamma, beta, bn, mode):
    N, H = z.shape
    G = H // 128
    NB = N // bn
    gb = jnp.stack([gamma, beta])

    def stats(ps_ref, pq_ref):
        s = jnp.sum(ps_ref[...], axis=(0, 1))
        q = jnp.sum(pq_ref[...], axis=(0, 1))
        mu = s / N
        var = q / N - mu * mu
        return mu, lax.rsqrt(var + 1e-5)

    if mode == "first":
        def body(z_ref, ps_ref, pq_ref, gb_ref, out_ref):
            mu, inv = stats(ps_ref, pq_ref)
            zn = gb_ref[0] * (z_ref[...] - mu) * inv + gb_ref[1]
            r = jnp.maximum(zn, 0.0)
            for g in range(G):
                out_ref[g] = r[:, g * 128:(g + 1) * 128]

        return pl.pallas_call(
            body,
            grid=(NB,),
            in_specs=[
                pl.BlockSpec((bn, H), lambda i: (i, 0)),
                pl.BlockSpec((NB, 1, H), lambda i: (0, 0, 0)),
                pl.BlockSpec((NB, 1, H), lambda i: (0, 0, 0)),
                pl.BlockSpec((2, H), lambda i: (0, 0)),
            ],
            out_specs=pl.BlockSpec((G, bn, 128), lambda i: (0, i, 0)),
            out_shape=jax.ShapeDtypeStruct((G, N, 128), jnp.float32),
        )(z, psum, psumsq, gb)

    if mode == "mid":
        def body(z_ref, h_ref, ps_ref, pq_ref, gb_ref, out_ref):
            mu, inv = stats(ps_ref, pq_ref)
            zn = gb_ref[0] * (z_ref[...] - mu) * inv + gb_ref[1]
            r = jnp.maximum(zn, 0.0)
            for g in range(G):
                out_ref[g] = h_ref[g] + r[:, g * 128:(g + 1) * 128]

        return pl.pallas_call(
            body,
            grid=(NB,),
            in_specs=[
                pl.BlockSpec((bn, H), lambda i: (i, 0)),
                pl.BlockSpec((G, bn, 128), lambda i: (0, i, 0)),
                pl.BlockSpec((NB, 1, H), lambda i: (0, 0, 0)),
                pl.BlockSpec((NB, 1, H), lambda i: (0, 0, 0)),
                pl.BlockSpec((2, H), lambda i: (0, 0)),
            ],
            out_specs=pl.BlockSpec((G, bn, 128), lambda i: (0, i, 0)),
            out_shape=jax.ShapeDtypeStruct((G, N, 128), jnp.float32),
        )(z, h, psum, psumsq, gb)

    def body(z_ref, h_ref, ps_ref, pq_ref, gb_ref, out_ref):
        mu, inv = stats(ps_ref, pq_ref)
        zn = gb_ref[0] * (z_ref[...] - mu) * inv + gb_ref[1]
        r = jnp.maximum(zn, 0.0)
        hfull = jnp.concatenate([h_ref[g] for g in range(G)], axis=1)
        out_ref[...] = hfull + r

    return pl.pallas_call(
        body,
        grid=(NB,),
        in_specs=[
            pl.BlockSpec((bn, H), lambda i: (i, 0)),
            pl.BlockSpec((G, bn, 128), lambda i: (0, i, 0)),
            pl.BlockSpec((NB, 1, H), lambda i: (0, 0, 0)),
            pl.BlockSpec((NB, 1, H), lambda i: (0, 0, 0)),
            pl.BlockSpec((2, H), lambda i: (0, 0)),
        ],
        out_specs=pl.BlockSpec((bn, H), lambda i: (i, 0)),
        out_shape=jax.ShapeDtypeStruct((N, H), jnp.float32),
    )(z, h, psum, psumsq, gb)


def kernel(x, edge_index, W_rel0, b_rel0, W_root0, gamma0, beta0,
           W_rel, b_rel, W_root, gamma, beta):
    N, D = x.shape
    H = W_rel0.shape[1]
    G0 = D // 128
    G = H // 128
    E = edge_index.shape[1]
    bn = 1000

    src = edge_index[0].astype(jnp.int32)
    dst = edge_index[1].astype(jnp.int32)
    e_pad = ((E + _K - 1) // _K) * _K
    if e_pad != E:
        pad_pos = jnp.arange(E, e_pad, dtype=jnp.int32)
        src = jnp.concatenate([src, jnp.zeros((e_pad - E,), jnp.int32)])
        dst = jnp.concatenate([dst, N + pad_pos % 16])

    x_b = x.reshape(N, G0, 128).transpose(1, 0, 2)

    agg = _segment_sum_cols(x_b.reshape(G0 * N, 128), src, dst, G0, N)
    z, ps, pq = _tc_linear(agg.reshape(G0, N, 128), x_b,
                           W_rel0.reshape(G0, 128, H),
                           W_root0.reshape(G0, 128, H), b_rel0, bn)
    out = _tc_bn(z, None, ps, pq, gamma0, beta0, bn, "first")

    for i in range(4):
        agg = _segment_sum_cols(out.reshape(G * N, 128), src, dst, G, N)
        z, ps, pq = _tc_linear(agg.reshape(G, N, 128), out,
                               W_rel[i].reshape(G, 128, H),
                               W_root[i].reshape(G, 128, H), b_rel[i], bn)
        mode = "last" if i == 3 else "mid"
        out = _tc_bn(z, out, ps, pq, gamma[i], beta[i], bn, mode)

    return out


# speedup vs baseline: 1.7294x; 1.0841x over previous
import functools

import jax
import jax.numpy as jnp
from jax import lax
from jax.experimental import pallas as pl
from jax.experimental.pallas import tpu as pltpu
from jax.experimental.pallas import tpu_sc as plsc

_LANES = 16
_K = 128


_BATCH = 8


@functools.lru_cache(maxsize=None)
def _make_scatter(G, N, nblk):
    assert N % 16 == 0 and G % 2 == 0
    niter = (nblk + 15) // 16
    z_rows = (((N + _LANES) // _LANES) // 8 + 1) * 8
    n_pad = 16 * z_rows
    per_sc = G // 2
    w_rows = (N // 16) // 8 * 8
    w_tail = N - 16 * w_rows
    assert w_tail % 8 == 0

    mesh = plsc.VectorSubcoreMesh(core_axis_name="c", subcore_axis_name="s")

    @functools.partial(
        pl.kernel,
        mesh=mesh,
        out_type=jax.ShapeDtypeStruct((G * N, 128), jnp.float32),
        scratch_types=[
            pltpu.VMEM((_K,), jnp.int32),
            pltpu.VMEM((_K,), jnp.int32),
            pltpu.VMEM((_K,), jnp.int32),
            pltpu.VMEM((_K, 128), jnp.float32),
            pltpu.VMEM((64, 128), jnp.float32),
            pltpu.VMEM_SHARED((n_pad, 128), jnp.float32),
            pltpu.SemaphoreType.DMA,
            pltpu.SemaphoreType.DMA,
        ],
    )
    def scatter_kernel(h_hbm, src_hbm, dst_hbm, out_hbm,
                       sidx, didx, gidx, rows, zbuf, acc, sem, sem2):
        scid = lax.axis_index("c")
        sid = lax.axis_index("s")

        zv = jnp.zeros((_LANES,), jnp.float32)

        def _zrow(r, carry):
            for j in range(128 // _LANES):
                zbuf[r, pl.ds(j * _LANES, _LANES)] = zv
            return carry

        lax.fori_loop(0, 64, _zrow, 0)
        z_full, z_tail = z_rows // 64, z_rows % 64
        assert z_tail % 8 == 0

        for p in range(per_sc):
            g = scid * per_sc + p
            goff = g * N

            def _zcopy(i, carry):
                pltpu.sync_copy(
                    zbuf, acc.at[pl.ds(sid * z_rows + i * 64, 64)])
                return carry

            lax.fori_loop(0, z_full, _zcopy, 0)
            if z_tail:
                pltpu.sync_copy(
                    zbuf.at[pl.ds(0, z_tail)],
                    acc.at[pl.ds(sid * z_rows + z_full * 64, z_tail)])
            plsc.subcore_barrier()

            def _blk(i, carry):
                b = sid + i * 16

                @pl.when(b < nblk)
                def _():
                    e0 = b * _K
                    c1 = pltpu.async_copy(
                        src_hbm.at[pl.ds(e0, _K)], sidx, sem)
                    c2 = pltpu.async_copy(
                        dst_hbm.at[pl.ds(e0, _K)], didx, sem2)
                    c1.wait()
                    c2.wait()
                    for c in range(_K // _LANES):
                        sl = pl.ds(c * _LANES, _LANES)
                        gidx[sl] = sidx[sl] + goff
                    pltpu.async_copy(h_hbm.at[gidx], rows, sem).wait()
                    pltpu.sync_copy(rows, acc.at[didx], add=True)

                return carry

            lax.fori_loop(0, niter, _blk, 0)
            plsc.subcore_barrier()

            r0 = sid * w_rows
            pltpu.sync_copy(acc.at[pl.ds(r0, w_rows)],
                            out_hbm.at[pl.ds(goff + r0, w_rows)])
            if w_tail:
                @pl.when(sid == 0)
                def _():
                    pltpu.sync_copy(
                        acc.at[pl.ds(16 * w_rows, w_tail)],
                        out_hbm.at[pl.ds(goff + 16 * w_rows, w_tail)])
            plsc.subcore_barrier()

    return scatter_kernel


def _segment_sum_cols(h_flat, src, dst, G, N):
    return _make_scatter(G, N, src.shape[0] // _K)(h_flat, src, dst)


def _tc_linear(agg, h, Wr, Wro, br, bn):
    G, N, _ = agg.shape
    H = Wr.shape[2]
    NB = N // bn

    def body(agg_ref, h_ref, wr_ref, wro_ref, br_ref, z_ref, ps_ref, pq_ref):
        z = jnp.broadcast_to(br_ref[...], (bn, H)).astype(jnp.float32)
        for g in range(G):
            z = z + jnp.dot(agg_ref[g], wr_ref[g],
                            preferred_element_type=jnp.float32)
            z = z + jnp.dot(h_ref[g], wro_ref[g],
                            preferred_element_type=jnp.float32)
        z_ref[...] = z
        ps_ref[...] = jnp.sum(z, axis=0).reshape(1, 1, H)
        pq_ref[...] = jnp.sum(z * z, axis=0).reshape(1, 1, H)

    return pl.pallas_call(
        body,
        grid=(NB,),
        in_specs=[
            pl.BlockSpec((G, bn, 128), lambda i: (0, i, 0)),
            pl.BlockSpec((G, bn, 128), lambda i: (0, i, 0)),
            pl.BlockSpec((G, 128, H), lambda i: (0, 0, 0)),
            pl.BlockSpec((G, 128, H), lambda i: (0, 0, 0)),
            pl.BlockSpec((1, H), lambda i: (0, 0)),
        ],
        out_specs=[
            pl.BlockSpec((bn, H), lambda i: (i, 0)),
            pl.BlockSpec((1, 1, H), lambda i: (i, 0, 0)),
            pl.BlockSpec((1, 1, H), lambda i: (i, 0, 0)),
        ],
        out_shape=[
            jax.ShapeDtypeStruct((N, H), jnp.float32),
            jax.ShapeDtypeStruct((NB, 1, H), jnp.float32),
            jax.ShapeDtypeStruct((NB, 1, H), jnp.float32),
        ],
    )(agg, h, Wr, Wro, br.reshape(1, H))


def _tc_bn(z, h, psum, psumsq, gamma, beta, bn, mode):
    N, H = z.shape
    G = H // 128
    NB = N // bn
    gb = jnp.stack([gamma, beta])

    def stats(ps_ref, pq_ref):
        s = jnp.sum(ps_ref[...], axis=(0, 1))
        q = jnp.sum(pq_ref[...], axis=(0, 1))
        mu = s / N
        var = q / N - mu * mu
        return mu, lax.rsqrt(var + 1e-5)

    if mode == "first":
        def body(z_ref, ps_ref, pq_ref, gb_ref, out_ref):
            mu, inv = stats(ps_ref, pq_ref)
            zn = gb_ref[0] * (z_ref[...] - mu) * inv + gb_ref[1]
            r = jnp.maximum(zn, 0.0)
            for g in range(G):
                out_ref[g] = r[:, g * 128:(g + 1) * 128]

        return pl.pallas_call(
            body,
            grid=(NB,),
            in_specs=[
                pl.BlockSpec((bn, H), lambda i: (i, 0)),
                pl.BlockSpec((NB, 1, H), lambda i: (0, 0, 0)),
                pl.BlockSpec((NB, 1, H), lambda i: (0, 0, 0)),
                pl.BlockSpec((2, H), lambda i: (0, 0)),
            ],
            out_specs=pl.BlockSpec((G, bn, 128), lambda i: (0, i, 0)),
            out_shape=jax.ShapeDtypeStruct((G, N, 128), jnp.float32),
        )(z, psum, psumsq, gb)

    if mode == "mid":
        def body(z_ref, h_ref, ps_ref, pq_ref, gb_ref, out_ref):
            mu, inv = stats(ps_ref, pq_ref)
            zn = gb_ref[0] * (z_ref[...] - mu) * inv + gb_ref[1]
            r = jnp.maximum(zn, 0.0)
            for g in range(G):
                out_ref[g] = h_ref[g] + r[:, g * 128:(g + 1) * 128]

        return pl.pallas_call(
            body,
            grid=(NB,),
            in_specs=[
                pl.BlockSpec((bn, H), lambda i: (i, 0)),
                pl.BlockSpec((G, bn, 128), lambda i: (0, i, 0)),
                pl.BlockSpec((NB, 1, H), lambda i: (0, 0, 0)),
                pl.BlockSpec((NB, 1, H), lambda i: (0, 0, 0)),
                pl.BlockSpec((2, H), lambda i: (0, 0)),
            ],
            out_specs=pl.BlockSpec((G, bn, 128), lambda i: (0, i, 0)),
            out_shape=jax.ShapeDtypeStruct((G, N, 128), jnp.float32),
        )(z, h, psum, psumsq, gb)

    def body(z_ref, h_ref, ps_ref, pq_ref, gb_ref, out_ref):
        mu, inv = stats(ps_ref, pq_ref)
        zn = gb_ref[0] * (z_ref[...] - mu) * inv + gb_ref[1]
        r = jnp.maximum(zn, 0.0)
        hfull = jnp.concatenate([h_ref[g] for g in range(G)], axis=1)
        out_ref[...] = hfull + r

    return pl.pallas_call(
        body,
        grid=(NB,),
        in_specs=[
            pl.BlockSpec((bn, H), lambda i: (i, 0)),
            pl.BlockSpec((G, bn, 128), lambda i: (0, i, 0)),
            pl.BlockSpec((NB, 1, H), lambda i: (0, 0, 0)),
            pl.BlockSpec((NB, 1, H), lambda i: (0, 0, 0)),
            pl.BlockSpec((2, H), lambda i: (0, 0)),
        ],
        out_specs=pl.BlockSpec((bn, H), lambda i: (i, 0)),
        out_shape=jax.ShapeDtypeStruct((N, H), jnp.float32),
    )(z, h, psum, psumsq, gb)


def kernel(x, edge_index, W_rel0, b_rel0, W_root0, gamma0, beta0,
           W_rel, b_rel, W_root, gamma, beta):
    N, D = x.shape
    H = W_rel0.shape[1]
    G0 = D // 128
    G = H // 128
    E = edge_index.shape[1]
    bn = 1000

    src = edge_index[0].astype(jnp.int32)
    dst = edge_index[1].astype(jnp.int32)
    e_pad = ((E + _K - 1) // _K) * _K
    if e_pad != E:
        pad_pos = jnp.arange(E, e_pad, dtype=jnp.int32)
        src = jnp.concatenate([src, jnp.zeros((e_pad - E,), jnp.int32)])
        dst = jnp.concatenate([dst, N + pad_pos % 16])

    x_b = x.reshape(N, G0, 128).transpose(1, 0, 2)

    agg = _segment_sum_cols(x_b.reshape(G0 * N, 128), src, dst, G0, N)
    z, ps, pq = _tc_linear(agg.reshape(G0, N, 128), x_b,
                           W_rel0.reshape(G0, 128, H),
                           W_root0.reshape(G0, 128, H), b_rel0, bn)
    out = _tc_bn(z, None, ps, pq, gamma0, beta0, bn, "first")

    for i in range(4):
        agg = _segment_sum_cols(out.reshape(G * N, 128), src, dst, G, N)
        z, ps, pq = _tc_linear(agg.reshape(G, N, 128), out,
                               W_rel[i].reshape(G, 128, H),
                               W_root[i].reshape(G, 128, H), b_rel[i], bn)
        mode = "last" if i == 3 else "mid"
        out = _tc_bn(z, out, ps, pq, gamma[i], beta[i], bn, mode)

    return out


# speedup vs baseline: 2.1196x; 1.2256x over previous
import functools

import jax
import jax.numpy as jnp
from jax import lax
from jax.experimental import pallas as pl
from jax.experimental.pallas import tpu as pltpu
from jax.experimental.pallas import tpu_sc as plsc

_LANES = 16
_K = 128


_BATCH = 8


@functools.lru_cache(maxsize=None)
def _make_scatter(G, N, nblk):
    assert N % 16 == 0 and G % 2 == 0
    niter = (nblk + 15) // 16
    z_rows = (((N + _LANES) // _LANES) // 8 + 1) * 8
    n_pad = 16 * z_rows
    per_sc = G // 2
    w_rows = (N // 16) // 8 * 8
    w_tail = N - 16 * w_rows
    assert w_tail % 8 == 0

    mesh = plsc.VectorSubcoreMesh(core_axis_name="c", subcore_axis_name="s")

    @functools.partial(
        pl.kernel,
        mesh=mesh,
        out_type=jax.ShapeDtypeStruct((G * N, 128), jnp.float32),
        scratch_types=[
            pltpu.VMEM((_K,), jnp.int32),
            pltpu.VMEM((_K,), jnp.int32),
            pltpu.VMEM((_K,), jnp.int32),
            pltpu.VMEM((_K, 128), jnp.float32),
            pltpu.VMEM((_K,), jnp.int32),
            pltpu.VMEM((_K,), jnp.int32),
            pltpu.VMEM((_K,), jnp.int32),
            pltpu.VMEM((_K, 128), jnp.float32),
            pltpu.VMEM((64, 128), jnp.float32),
            pltpu.VMEM_SHARED((n_pad, 128), jnp.float32),
            pltpu.SemaphoreType.DMA,
            pltpu.SemaphoreType.DMA,
            pltpu.SemaphoreType.DMA,
            pltpu.SemaphoreType.DMA,
            pltpu.SemaphoreType.DMA,
            pltpu.SemaphoreType.DMA,
        ],
    )
    def scatter_kernel(h_hbm, src_hbm, dst_hbm, out_hbm,
                       sidxa, didxa, gidxa, rowsa,
                       sidxb, didxb, gidxb, rowsb, zbuf, acc,
                       sa1, sa2, sa3, sb1, sb2, sb3):
        scid = lax.axis_index("c")
        sid = lax.axis_index("s")

        zv = jnp.zeros((_LANES,), jnp.float32)

        def _zrow(r, carry):
            for j in range(128 // _LANES):
                zbuf[r, pl.ds(j * _LANES, _LANES)] = zv
            return carry

        lax.fori_loop(0, 64, _zrow, 0)
        z_full, z_tail = z_rows // 64, z_rows % 64
        assert z_tail % 8 == 0

        for p in range(per_sc):
            g = scid * per_sc + p
            goff = g * N

            def _zcopy(i, carry):
                pltpu.sync_copy(
                    zbuf, acc.at[pl.ds(sid * z_rows + i * 64, 64)])
                return carry

            lax.fori_loop(0, z_full, _zcopy, 0)
            if z_tail:
                pltpu.sync_copy(
                    zbuf.at[pl.ds(0, z_tail)],
                    acc.at[pl.ds(sid * z_rows + z_full * 64, z_tail)])
            plsc.subcore_barrier()

            bufs = ((sidxa, didxa, gidxa, rowsa, sa1, sa2, sa3),
                    (sidxb, didxb, gidxb, rowsb, sb1, sb2, sb3))

            def _blk2(m, carry):
                bs = (sid + 32 * m, sid + 32 * m + 16)

                for k in range(2):
                    sidx, didx, gidx, rows, s1, s2, s3 = bufs[k]

                    @pl.when(bs[k] < nblk)
                    def _():
                        e0 = bs[k] * _K
                        pltpu.async_copy(src_hbm.at[pl.ds(e0, _K)], sidx, s1)
                        pltpu.async_copy(dst_hbm.at[pl.ds(e0, _K)], didx, s2)

                for k in range(2):
                    sidx, didx, gidx, rows, s1, s2, s3 = bufs[k]

                    @pl.when(bs[k] < nblk)
                    def _():
                        e0 = bs[k] * _K
                        pltpu.make_async_copy(
                            src_hbm.at[pl.ds(e0, _K)], sidx, s1).wait()
                        for c in range(_K // _LANES):
                            sl = pl.ds(c * _LANES, _LANES)
                            gidx[sl] = sidx[sl] + goff
                        pltpu.async_copy(h_hbm.at[gidx], rows, s3)

                for k in range(2):
                    sidx, didx, gidx, rows, s1, s2, s3 = bufs[k]

                    @pl.when(bs[k] < nblk)
                    def _():
                        e0 = bs[k] * _K
                        pltpu.make_async_copy(
                            h_hbm.at[gidx], rows, s3).wait()
                        pltpu.make_async_copy(
                            dst_hbm.at[pl.ds(e0, _K)], didx, s2).wait()
                        pltpu.sync_copy(rows, acc.at[didx], add=True)

                return carry

            lax.fori_loop(0, (niter + 1) // 2, _blk2, 0)
            plsc.subcore_barrier()

            r0 = sid * w_rows
            pltpu.sync_copy(acc.at[pl.ds(r0, w_rows)],
                            out_hbm.at[pl.ds(goff + r0, w_rows)])
            if w_tail:
                @pl.when(sid == 0)
                def _():
                    pltpu.sync_copy(
                        acc.at[pl.ds(16 * w_rows, w_tail)],
                        out_hbm.at[pl.ds(goff + 16 * w_rows, w_tail)])
            plsc.subcore_barrier()

    return scatter_kernel


def _segment_sum_cols(h_flat, src, dst, G, N):
    return _make_scatter(G, N, src.shape[0] // _K)(h_flat, src, dst)


def _tc_linear(agg, h, Wr, Wro, br, bn):
    G, N, _ = agg.shape
    H = Wr.shape[2]
    NB = N // bn

    def body(agg_ref, h_ref, wr_ref, wro_ref, br_ref, z_ref, ps_ref, pq_ref):
        z = jnp.broadcast_to(br_ref[...], (bn, H)).astype(jnp.float32)
        for g in range(G):
            z = z + jnp.dot(agg_ref[g], wr_ref[g],
                            preferred_element_type=jnp.float32)
            z = z + jnp.dot(h_ref[g], wro_ref[g],
                            preferred_element_type=jnp.float32)
        z_ref[...] = z
        ps_ref[...] = jnp.sum(z, axis=0).reshape(1, 1, H)
        pq_ref[...] = jnp.sum(z * z, axis=0).reshape(1, 1, H)

    return pl.pallas_call(
        body,
        grid=(NB,),
        in_specs=[
            pl.BlockSpec((G, bn, 128), lambda i: (0, i, 0)),
            pl.BlockSpec((G, bn, 128), lambda i: (0, i, 0)),
            pl.BlockSpec((G, 128, H), lambda i: (0, 0, 0)),
            pl.BlockSpec((G, 128, H), lambda i: (0, 0, 0)),
            pl.BlockSpec((1, H), lambda i: (0, 0)),
        ],
        out_specs=[
            pl.BlockSpec((bn, H), lambda i: (i, 0)),
            pl.BlockSpec((1, 1, H), lambda i: (i, 0, 0)),
            pl.BlockSpec((1, 1, H), lambda i: (i, 0, 0)),
        ],
        out_shape=[
            jax.ShapeDtypeStruct((N, H), jnp.float32),
            jax.ShapeDtypeStruct((NB, 1, H), jnp.float32),
            jax.ShapeDtypeStruct((NB, 1, H), jnp.float32),
        ],
    )(agg, h, Wr, Wro, br.reshape(1, H))


def _tc_bn(z, h, psum, psumsq, gamma, beta, bn, mode):
    N, H = z.shape
    G = H // 128
    NB = N // bn
    gb = jnp.stack([gamma, beta])

    def stats(ps_ref, pq_ref):
        s = jnp.sum(ps_ref[...], axis=(0, 1))
        q = jnp.sum(pq_ref[...], axis=(0, 1))
        mu = s / N
        var = q / N - mu * mu
        return mu, lax.rsqrt(var + 1e-5)

    if mode == "first":
        def body(z_ref, ps_ref, pq_ref, gb_ref, out_ref):
            mu, inv = stats(ps_ref, pq_ref)
            zn = gb_ref[0] * (z_ref[...] - mu) * inv + gb_ref[1]
            r = jnp.maximum(zn, 0.0)
            for g in range(G):
                out_ref[g] = r[:, g * 128:(g + 1) * 128]

        return pl.pallas_call(
            body,
            grid=(NB,),
            in_specs=[
                pl.BlockSpec((bn, H), lambda i: (i, 0)),
                pl.BlockSpec((NB, 1, H), lambda i: (0, 0, 0)),
                pl.BlockSpec((NB, 1, H), lambda i: (0, 0, 0)),
                pl.BlockSpec((2, H), lambda i: (0, 0)),
            ],
            out_specs=pl.BlockSpec((G, bn, 128), lambda i: (0, i, 0)),
            out_shape=jax.ShapeDtypeStruct((G, N, 128), jnp.float32),
        )(z, psum, psumsq, gb)

    if mode == "mid":
        def body(z_ref, h_ref, ps_ref, pq_ref, gb_ref, out_ref):
            mu, inv = stats(ps_ref, pq_ref)
            zn = gb_ref[0] * (z_ref[...] - mu) * inv + gb_ref[1]
            r = jnp.maximum(zn, 0.0)
            for g in range(G):
                out_ref[g] = h_ref[g] + r[:, g * 128:(g + 1) * 128]

        return pl.pallas_call(
            body,
            grid=(NB,),
            in_specs=[
                pl.BlockSpec((bn, H), lambda i: (i, 0)),
                pl.BlockSpec((G, bn, 128), lambda i: (0, i, 0)),
                pl.BlockSpec((NB, 1, H), lambda i: (0, 0, 0)),
                pl.BlockSpec((NB, 1, H), lambda i: (0, 0, 0)),
                pl.BlockSpec((2, H), lambda i: (0, 0)),
            ],
            out_specs=pl.BlockSpec((G, bn, 128), lambda i: (0, i, 0)),
            out_shape=jax.ShapeDtypeStruct((G, N, 128), jnp.float32),
        )(z, h, psum, psumsq, gb)

    def body(z_ref, h_ref, ps_ref, pq_ref, gb_ref, out_ref):
        mu, inv = stats(ps_ref, pq_ref)
        zn = gb_ref[0] * (z_ref[...] - mu) * inv + gb_ref[1]
        r = jnp.maximum(zn, 0.0)
        hfull = jnp.concatenate([h_ref[g] for g in range(G)], axis=1)
        out_ref[...] = hfull + r

    return pl.pallas_call(
        body,
        grid=(NB,),
        in_specs=[
            pl.BlockSpec((bn, H), lambda i: (i, 0)),
            pl.BlockSpec((G, bn, 128), lambda i: (0, i, 0)),
            pl.BlockSpec((NB, 1, H), lambda i: (0, 0, 0)),
            pl.BlockSpec((NB, 1, H), lambda i: (0, 0, 0)),
            pl.BlockSpec((2, H), lambda i: (0, 0)),
        ],
        out_specs=pl.BlockSpec((bn, H), lambda i: (i, 0)),
        out_shape=jax.ShapeDtypeStruct((N, H), jnp.float32),
    )(z, h, psum, psumsq, gb)


def kernel(x, edge_index, W_rel0, b_rel0, W_root0, gamma0, beta0,
           W_rel, b_rel, W_root, gamma, beta):
    N, D = x.shape
    H = W_rel0.shape[1]
    G0 = D // 128
    G = H // 128
    E = edge_index.shape[1]
    bn = 1000

    src = edge_index[0].astype(jnp.int32)
    dst = edge_index[1].astype(jnp.int32)
    e_pad = ((E + _K - 1) // _K) * _K
    if e_pad != E:
        pad_pos = jnp.arange(E, e_pad, dtype=jnp.int32)
        src = jnp.concatenate([src, jnp.zeros((e_pad - E,), jnp.int32)])
        dst = jnp.concatenate([dst, N + pad_pos % 16])

    x_b = x.reshape(N, G0, 128).transpose(1, 0, 2)

    agg = _segment_sum_cols(x_b.reshape(G0 * N, 128), src, dst, G0, N)
    z, ps, pq = _tc_linear(agg.reshape(G0, N, 128), x_b,
                           W_rel0.reshape(G0, 128, H),
                           W_root0.reshape(G0, 128, H), b_rel0, bn)
    out = _tc_bn(z, None, ps, pq, gamma0, beta0, bn, "first")

    for i in range(4):
        agg = _segment_sum_cols(out.reshape(G * N, 128), src, dst, G, N)
        z, ps, pq = _tc_linear(agg.reshape(G, N, 128), out,
                               W_rel[i].reshape(G, 128, H),
                               W_root[i].reshape(G, 128, H), b_rel[i], bn)
        mode = "last" if i == 3 else "mid"
        out = _tc_bn(z, out, ps, pq, gamma[i], beta[i], bn, mode)

    return out


# speedup vs baseline: 2.2566x; 1.0646x over previous
import functools

import jax
import jax.numpy as jnp
from jax import lax
from jax.experimental import pallas as pl
from jax.experimental.pallas import tpu as pltpu
from jax.experimental.pallas import tpu_sc as plsc

_LANES = 16
_K = 128


_BATCH = 8


@functools.lru_cache(maxsize=None)
def _make_scatter(G, N, nblk):
    assert N % 16 == 0 and G % 2 == 0
    niter = (nblk + 15) // 16
    z_rows = (((N + _LANES) // _LANES) // 8 + 1) * 8
    n_pad = 16 * z_rows
    per_sc = G // 2
    w_rows = (N // 16) // 8 * 8
    w_tail = N - 16 * w_rows
    assert w_tail % 8 == 0

    mesh = plsc.VectorSubcoreMesh(core_axis_name="c", subcore_axis_name="s")

    @functools.partial(
        pl.kernel,
        mesh=mesh,
        out_type=jax.ShapeDtypeStruct((G * N, 128), jnp.float32),
        scratch_types=[
            pltpu.VMEM((_K,), jnp.int32),
            pltpu.VMEM((_K,), jnp.int32),
            pltpu.VMEM((_K, 128), jnp.float32),
            pltpu.VMEM((_K,), jnp.int32),
            pltpu.VMEM((_K,), jnp.int32),
            pltpu.VMEM((_K, 128), jnp.float32),
            pltpu.VMEM((_K,), jnp.int32),
            pltpu.VMEM((_K,), jnp.int32),
            pltpu.VMEM((_K, 128), jnp.float32),
            pltpu.VMEM_SHARED((n_pad, 128), jnp.float32),
            pltpu.SemaphoreType.DMA,
            pltpu.SemaphoreType.DMA,
            pltpu.SemaphoreType.DMA,
            pltpu.SemaphoreType.DMA,
            pltpu.SemaphoreType.DMA,
            pltpu.SemaphoreType.DMA,
            pltpu.SemaphoreType.DMA,
            pltpu.SemaphoreType.DMA,
            pltpu.SemaphoreType.DMA,
        ],
    )
    def scatter_kernel(h_hbm, src_hbm, dst_hbm, out_hbm,
                       gixa, didxa, rowsa, gixb, didxb, rowsb,
                       gixc, didxc, rowsc, acc,
                       sa1, sa2, sa3, sb1, sb2, sb3, sc1, sc2, sc3):
        scid = lax.axis_index("c")
        sid = lax.axis_index("s")

        zv = jnp.zeros((_LANES,), jnp.float32)

        def _zrow(r, carry):
            for j in range(128 // _LANES):
                rowsa[r, pl.ds(j * _LANES, _LANES)] = zv
            return carry

        z_full, z_tail = z_rows // _K, z_rows % _K
        assert z_tail % 8 == 0

        for p in range(per_sc):
            g = scid * per_sc + p
            goff = g * N

            lax.fori_loop(0, _K, _zrow, 0)

            def _zcopy(i, carry):
                pltpu.sync_copy(
                    rowsa, acc.at[pl.ds(sid * z_rows + i * _K, _K)])
                return carry

            lax.fori_loop(0, z_full, _zcopy, 0)
            if z_tail:
                pltpu.sync_copy(
                    rowsa.at[pl.ds(0, z_tail)],
                    acc.at[pl.ds(sid * z_rows + z_full * _K, z_tail)])
            plsc.subcore_barrier()

            bufs = ((gixa, didxa, rowsa, sa1, sa2, sa3),
                    (gixb, didxb, rowsb, sb1, sb2, sb3),
                    (gixc, didxc, rowsc, sc1, sc2, sc3))
            depth = len(bufs)

            def _blkn(m, carry):
                bs = tuple(sid + 16 * (depth * m + k) for k in range(depth))

                for k in range(depth):
                    gix, didx, rows, s1, s2, s3 = bufs[k]

                    @pl.when(bs[k] < nblk)
                    def _():
                        e0 = bs[k] * _K
                        pltpu.async_copy(src_hbm.at[pl.ds(e0, _K)], gix, s1)
                        pltpu.async_copy(dst_hbm.at[pl.ds(e0, _K)], didx, s2)

                for k in range(depth):
                    gix, didx, rows, s1, s2, s3 = bufs[k]

                    @pl.when(bs[k] < nblk)
                    def _():
                        e0 = bs[k] * _K
                        pltpu.make_async_copy(
                            src_hbm.at[pl.ds(e0, _K)], gix, s1).wait()
                        for c in range(_K // _LANES):
                            sl = pl.ds(c * _LANES, _LANES)
                            gix[sl] = gix[sl] + goff
                        pltpu.async_copy(h_hbm.at[gix], rows, s3)

                for k in range(depth):
                    gix, didx, rows, s1, s2, s3 = bufs[k]

                    @pl.when(bs[k] < nblk)
                    def _():
                        e0 = bs[k] * _K
                        pltpu.make_async_copy(
                            h_hbm.at[gix], rows, s3).wait()
                        pltpu.make_async_copy(
                            dst_hbm.at[pl.ds(e0, _K)], didx, s2).wait()
                        pltpu.sync_copy(rows, acc.at[didx], add=True)

                return carry

            lax.fori_loop(0, (niter + depth - 1) // depth, _blkn, 0)
            plsc.subcore_barrier()

            r0 = sid * w_rows
            pltpu.sync_copy(acc.at[pl.ds(r0, w_rows)],
                            out_hbm.at[pl.ds(goff + r0, w_rows)])
            if w_tail:
                @pl.when(sid == 0)
                def _():
                    pltpu.sync_copy(
                        acc.at[pl.ds(16 * w_rows, w_tail)],
                        out_hbm.at[pl.ds(goff + 16 * w_rows, w_tail)])
            plsc.subcore_barrier()

    return scatter_kernel


def _segment_sum_cols(h_flat, src, dst, G, N):
    return _make_scatter(G, N, src.shape[0] // _K)(h_flat, src, dst)


def _tc_linear(agg, h, Wr, Wro, br, bn):
    G, N, _ = agg.shape
    H = Wr.shape[2]
    NB = N // bn

    def body(agg_ref, h_ref, wr_ref, wro_ref, br_ref, z_ref, ps_ref, pq_ref):
        z = jnp.broadcast_to(br_ref[...], (bn, H)).astype(jnp.float32)
        for g in range(G):
            z = z + jnp.dot(agg_ref[g], wr_ref[g],
                            preferred_element_type=jnp.float32)
            z = z + jnp.dot(h_ref[g], wro_ref[g],
                            preferred_element_type=jnp.float32)
        z_ref[...] = z
        ps_ref[...] = jnp.sum(z, axis=0).reshape(1, 1, H)
        pq_ref[...] = jnp.sum(z * z, axis=0).reshape(1, 1, H)

    return pl.pallas_call(
        body,
        grid=(NB,),
        in_specs=[
            pl.BlockSpec((G, bn, 128), lambda i: (0, i, 0)),
            pl.BlockSpec((G, bn, 128), lambda i: (0, i, 0)),
            pl.BlockSpec((G, 128, H), lambda i: (0, 0, 0)),
            pl.BlockSpec((G, 128, H), lambda i: (0, 0, 0)),
            pl.BlockSpec((1, H), lambda i: (0, 0)),
        ],
        out_specs=[
            pl.BlockSpec((bn, H), lambda i: (i, 0)),
            pl.BlockSpec((1, 1, H), lambda i: (i, 0, 0)),
            pl.BlockSpec((1, 1, H), lambda i: (i, 0, 0)),
        ],
        out_shape=[
            jax.ShapeDtypeStruct((N, H), jnp.float32),
            jax.ShapeDtypeStruct((NB, 1, H), jnp.float32),
            jax.ShapeDtypeStruct((NB, 1, H), jnp.float32),
        ],
    )(agg, h, Wr, Wro, br.reshape(1, H))


def _tc_bn(z, h, psum, psumsq, gamma, beta, bn, mode):
    N, H = z.shape
    G = H // 128
    NB = N // bn
    gb = jnp.stack([gamma, beta])

    def stats(ps_ref, pq_ref):
        s = jnp.sum(ps_ref[...], axis=(0, 1))
        q = jnp.sum(pq_ref[...], axis=(0, 1))
        mu = s / N
        var = q / N - mu * mu
        return mu, lax.rsqrt(var + 1e-5)

    if mode == "first":
        def body(z_ref, ps_ref, pq_ref, gb_ref, out_ref):
            mu, inv = stats(ps_ref, pq_ref)
            zn = gb_ref[0] * (z_ref[...] - mu) * inv + gb_ref[1]
            r = jnp.maximum(zn, 0.0)
            for g in range(G):
                out_ref[g] = r[:, g * 128:(g + 1) * 128]

        return pl.pallas_call(
            body,
            grid=(NB,),
            in_specs=[
                pl.BlockSpec((bn, H), lambda i: (i, 0)),
                pl.BlockSpec((NB, 1, H), lambda i: (0, 0, 0)),
                pl.BlockSpec((NB, 1, H), lambda i: (0, 0, 0)),
                pl.BlockSpec((2, H), lambda i: (0, 0)),
            ],
            out_specs=pl.BlockSpec((G, bn, 128), lambda i: (0, i, 0)),
            out_shape=jax.ShapeDtypeStruct((G, N, 128), jnp.float32),
        )(z, psum, psumsq, gb)

    if mode == "mid":
        def body(z_ref, h_ref, ps_ref, pq_ref, gb_ref, out_ref):
            mu, inv = stats(ps_ref, pq_ref)
            zn = gb_ref[0] * (z_ref[...] - mu) * inv + gb_ref[1]
            r = jnp.maximum(zn, 0.0)
            for g in range(G):
                out_ref[g] = h_ref[g] + r[:, g * 128:(g + 1) * 128]

        return pl.pallas_call(
            body,
            grid=(NB,),
            in_specs=[
                pl.BlockSpec((bn, H), lambda i: (i, 0)),
                pl.BlockSpec((G, bn, 128), lambda i: (0, i, 0)),
                pl.BlockSpec((NB, 1, H), lambda i: (0, 0, 0)),
                pl.BlockSpec((NB, 1, H), lambda i: (0, 0, 0)),
                pl.BlockSpec((2, H), lambda i: (0, 0)),
            ],
            out_specs=pl.BlockSpec((G, bn, 128), lambda i: (0, i, 0)),
            out_shape=jax.ShapeDtypeStruct((G, N, 128), jnp.float32),
        )(z, h, psum, psumsq, gb)

    def body(z_ref, h_ref, ps_ref, pq_ref, gb_ref, out_ref):
        mu, inv = stats(ps_ref, pq_ref)
        zn = gb_ref[0] * (z_ref[...] - mu) * inv + gb_ref[1]
        r = jnp.maximum(zn, 0.0)
        hfull = jnp.concatenate([h_ref[g] for g in range(G)], axis=1)
        out_ref[...] = hfull + r

    return pl.pallas_call(
        body,
        grid=(NB,),
        in_specs=[
            pl.BlockSpec((bn, H), lambda i: (i, 0)),
            pl.BlockSpec((G, bn, 128), lambda i: (0, i, 0)),
            pl.BlockSpec((NB, 1, H), lambda i: (0, 0, 0)),
            pl.BlockSpec((NB, 1, H), lambda i: (0, 0, 0)),
            pl.BlockSpec((2, H), lambda i: (0, 0)),
        ],
        out_specs=pl.BlockSpec((bn, H), lambda i: (i, 0)),
        out_shape=jax.ShapeDtypeStruct((N, H), jnp.float32),
    )(z, h, psum, psumsq, gb)


def kernel(x, edge_index, W_rel0, b_rel0, W_root0, gamma0, beta0,
           W_rel, b_rel, W_root, gamma, beta):
    N, D = x.shape
    H = W_rel0.shape[1]
    G0 = D // 128
    G = H // 128
    E = edge_index.shape[1]
    bn = 1000

    src = edge_index[0].astype(jnp.int32)
    dst = edge_index[1].astype(jnp.int32)
    e_pad = ((E + _K - 1) // _K) * _K
    if e_pad != E:
        pad_pos = jnp.arange(E, e_pad, dtype=jnp.int32)
        src = jnp.concatenate([src, jnp.zeros((e_pad - E,), jnp.int32)])
        dst = jnp.concatenate([dst, N + pad_pos % 16])

    x_b = x.reshape(N, G0, 128).transpose(1, 0, 2)

    agg = _segment_sum_cols(x_b.reshape(G0 * N, 128), src, dst, G0, N)
    z, ps, pq = _tc_linear(agg.reshape(G0, N, 128), x_b,
                           W_rel0.reshape(G0, 128, H),
                           W_root0.reshape(G0, 128, H), b_rel0, bn)
    out = _tc_bn(z, None, ps, pq, gamma0, beta0, bn, "first")

    for i in range(4):
        agg = _segment_sum_cols(out.reshape(G * N, 128), src, dst, G, N)
        z, ps, pq = _tc_linear(agg.reshape(G, N, 128), out,
                               W_rel[i].reshape(G, 128, H),
                               W_root[i].reshape(G, 128, H), b_rel[i], bn)
        mode = "last" if i == 3 else "mid"
        out = _tc_bn(z, out, ps, pq, gamma[i], beta[i], bn, mode)

    return out


# speedup vs baseline: 2.2629x; 1.0028x over previous
import functools

import jax
import jax.numpy as jnp
from jax import lax
from jax.experimental import pallas as pl
from jax.experimental.pallas import tpu as pltpu
from jax.experimental.pallas import tpu_sc as plsc

_LANES = 16
_K = 128


_BATCH = 8


@functools.lru_cache(maxsize=None)
def _make_scatter(G, N, nblk):
    assert N % 16 == 0 and G % 2 == 0
    niter = (nblk + 15) // 16
    z_rows = (((N + _LANES) // _LANES) // 8 + 1) * 8
    n_pad = 16 * z_rows
    per_sc = G // 2
    w_rows = (N // 16) // 8 * 8
    w_tail = N - 16 * w_rows
    assert w_tail % 8 == 0

    mesh = plsc.VectorSubcoreMesh(core_axis_name="c", subcore_axis_name="s")

    @functools.partial(
        pl.kernel,
        mesh=mesh,
        out_type=jax.ShapeDtypeStruct((G * N, 128), jnp.float32),
        scratch_types=[
            pltpu.VMEM((_K,), jnp.int32),
            pltpu.VMEM((_K,), jnp.int32),
            pltpu.VMEM((_K, 128), jnp.float32),
            pltpu.VMEM((_K,), jnp.int32),
            pltpu.VMEM((_K,), jnp.int32),
            pltpu.VMEM((_K, 128), jnp.float32),
            pltpu.VMEM((_K,), jnp.int32),
            pltpu.VMEM((_K,), jnp.int32),
            pltpu.VMEM((_K, 128), jnp.float32),
            pltpu.VMEM_SHARED((n_pad, 128), jnp.float32),
            pltpu.SemaphoreType.DMA,
            pltpu.SemaphoreType.DMA,
            pltpu.SemaphoreType.DMA,
            pltpu.SemaphoreType.DMA,
            pltpu.SemaphoreType.DMA,
            pltpu.SemaphoreType.DMA,
            pltpu.SemaphoreType.DMA,
            pltpu.SemaphoreType.DMA,
            pltpu.SemaphoreType.DMA,
        ],
    )
    def scatter_kernel(h_hbm, src_hbm, dst_hbm, out_hbm,
                       gixa, didxa, rowsa, gixb, didxb, rowsb,
                       gixc, didxc, rowsc, acc,
                       sa1, sa2, sa3, sb1, sb2, sb3, sc1, sc2, sc3):
        scid = lax.axis_index("c")
        sid = lax.axis_index("s")

        zv = jnp.zeros((_LANES,), jnp.float32)

        def _zrow(r, carry):
            for j in range(128 // _LANES):
                rowsa[r, pl.ds(j * _LANES, _LANES)] = zv
            return carry

        z_full, z_tail = z_rows // _K, z_rows % _K
        assert z_tail % 8 == 0

        for p in range(per_sc):
            g = scid * per_sc + p
            goff = g * N

            lax.fori_loop(0, _K, _zrow, 0)

            def _zcopy(i, carry):
                pltpu.sync_copy(
                    rowsa, acc.at[pl.ds(sid * z_rows + i * _K, _K)])
                return carry

            lax.fori_loop(0, z_full, _zcopy, 0)
            if z_tail:
                pltpu.sync_copy(
                    rowsa.at[pl.ds(0, z_tail)],
                    acc.at[pl.ds(sid * z_rows + z_full * _K, z_tail)])
            plsc.subcore_barrier()

            bufs = ((gixa, didxa, rowsa, sa1, sa2, sa3),
                    (gixb, didxb, rowsb, sb1, sb2, sb3),
                    (gixc, didxc, rowsc, sc1, sc2, sc3))
            depth = len(bufs)

            def _blkn(m, carry):
                bs = tuple(sid + 16 * (depth * m + k) for k in range(depth))

                for k in range(depth):
                    gix, didx, rows, s1, s2, s3 = bufs[k]

                    @pl.when(bs[k] < nblk)
                    def _():
                        e0 = bs[k] * _K
                        pltpu.async_copy(src_hbm.at[pl.ds(e0, _K)], gix, s1)
                        pltpu.async_copy(dst_hbm.at[pl.ds(e0, _K)], didx, s2)

                for k in range(depth):
                    gix, didx, rows, s1, s2, s3 = bufs[k]

                    @pl.when(bs[k] < nblk)
                    def _():
                        e0 = bs[k] * _K
                        pltpu.make_async_copy(
                            src_hbm.at[pl.ds(e0, _K)], gix, s1).wait()
                        for c in range(_K // _LANES):
                            sl = pl.ds(c * _LANES, _LANES)
                            gix[sl] = gix[sl] + goff
                        pltpu.async_copy(h_hbm.at[gix], rows, s3)

                for k in range(depth):
                    gix, didx, rows, s1, s2, s3 = bufs[k]

                    @pl.when(bs[k] < nblk)
                    def _():
                        e0 = bs[k] * _K
                        pltpu.make_async_copy(
                            h_hbm.at[gix], rows, s3).wait()
                        pltpu.make_async_copy(
                            dst_hbm.at[pl.ds(e0, _K)], didx, s2).wait()
                        pltpu.sync_copy(rows, acc.at[didx], add=True)

                return carry

            lax.fori_loop(0, (niter + depth - 1) // depth, _blkn, 0)
            plsc.subcore_barrier()

            r0 = sid * w_rows
            pltpu.sync_copy(acc.at[pl.ds(r0, w_rows)],
                            out_hbm.at[pl.ds(goff + r0, w_rows)])
            if w_tail:
                @pl.when(sid == 0)
                def _():
                    pltpu.sync_copy(
                        acc.at[pl.ds(16 * w_rows, w_tail)],
                        out_hbm.at[pl.ds(goff + 16 * w_rows, w_tail)])
            plsc.subcore_barrier()

    return scatter_kernel


def _segment_sum_cols(h_flat, src, dst, G, N):
    return _make_scatter(G, N, src.shape[0] // _K)(h_flat, src, dst)


def _tc_root(h, Wro, br, bn):
    G, N, _ = h.shape
    H = Wro.shape[2]
    NB = N // bn

    def body(h_ref, wro_ref, br_ref, z_ref):
        z = jnp.broadcast_to(br_ref[...], (bn, H)).astype(jnp.float32)
        for g in range(G):
            z = z + jnp.dot(h_ref[g], wro_ref[g],
                            preferred_element_type=jnp.float32)
        z_ref[...] = z

    return pl.pallas_call(
        body,
        grid=(NB,),
        in_specs=[
            pl.BlockSpec((G, bn, 128), lambda i: (0, i, 0)),
            pl.BlockSpec((G, 128, H), lambda i: (0, 0, 0)),
            pl.BlockSpec((1, H), lambda i: (0, 0)),
        ],
        out_specs=pl.BlockSpec((bn, H), lambda i: (i, 0)),
        out_shape=jax.ShapeDtypeStruct((N, H), jnp.float32),
    )(h, Wro, br.reshape(1, H))


def _tc_rel(agg, zr, Wr, bn):
    G, N, _ = agg.shape
    H = Wr.shape[2]
    NB = N // bn

    def body(agg_ref, zr_ref, wr_ref, z_ref, ps_ref, pq_ref):
        z = zr_ref[...]
        for g in range(G):
            z = z + jnp.dot(agg_ref[g], wr_ref[g],
                            preferred_element_type=jnp.float32)
        z_ref[...] = z
        ps_ref[...] = jnp.sum(z, axis=0).reshape(1, 1, H)
        pq_ref[...] = jnp.sum(z * z, axis=0).reshape(1, 1, H)

    return pl.pallas_call(
        body,
        grid=(NB,),
        in_specs=[
            pl.BlockSpec((G, bn, 128), lambda i: (0, i, 0)),
            pl.BlockSpec((bn, H), lambda i: (i, 0)),
            pl.BlockSpec((G, 128, H), lambda i: (0, 0, 0)),
        ],
        out_specs=[
            pl.BlockSpec((bn, H), lambda i: (i, 0)),
            pl.BlockSpec((1, 1, H), lambda i: (i, 0, 0)),
            pl.BlockSpec((1, 1, H), lambda i: (i, 0, 0)),
        ],
        out_shape=[
            jax.ShapeDtypeStruct((N, H), jnp.float32),
            jax.ShapeDtypeStruct((NB, 1, H), jnp.float32),
            jax.ShapeDtypeStruct((NB, 1, H), jnp.float32),
        ],
    )(agg, zr, Wr)


def _tc_bn(z, h, psum, psumsq, gamma, beta, bn, mode):
    N, H = z.shape
    G = H // 128
    NB = N // bn
    gb = jnp.stack([gamma, beta])

    def stats(ps_ref, pq_ref):
        s = jnp.sum(ps_ref[...], axis=(0, 1))
        q = jnp.sum(pq_ref[...], axis=(0, 1))
        mu = s / N
        var = q / N - mu * mu
        return mu, lax.rsqrt(var + 1e-5)

    if mode == "first":
        def body(z_ref, ps_ref, pq_ref, gb_ref, out_ref):
            mu, inv = stats(ps_ref, pq_ref)
            zn = gb_ref[0] * (z_ref[...] - mu) * inv + gb_ref[1]
            r = jnp.maximum(zn, 0.0)
            for g in range(G):
                out_ref[g] = r[:, g * 128:(g + 1) * 128]

        return pl.pallas_call(
            body,
            grid=(NB,),
            in_specs=[
                pl.BlockSpec((bn, H), lambda i: (i, 0)),
                pl.BlockSpec((NB, 1, H), lambda i: (0, 0, 0)),
                pl.BlockSpec((NB, 1, H), lambda i: (0, 0, 0)),
                pl.BlockSpec((2, H), lambda i: (0, 0)),
            ],
            out_specs=pl.BlockSpec((G, bn, 128), lambda i: (0, i, 0)),
            out_shape=jax.ShapeDtypeStruct((G, N, 128), jnp.float32),
        )(z, psum, psumsq, gb)

    if mode == "mid":
        def body(z_ref, h_ref, ps_ref, pq_ref, gb_ref, out_ref):
            mu, inv = stats(ps_ref, pq_ref)
            zn = gb_ref[0] * (z_ref[...] - mu) * inv + gb_ref[1]
            r = jnp.maximum(zn, 0.0)
            for g in range(G):
                out_ref[g] = h_ref[g] + r[:, g * 128:(g + 1) * 128]

        return pl.pallas_call(
            body,
            grid=(NB,),
            in_specs=[
                pl.BlockSpec((bn, H), lambda i: (i, 0)),
                pl.BlockSpec((G, bn, 128), lambda i: (0, i, 0)),
                pl.BlockSpec((NB, 1, H), lambda i: (0, 0, 0)),
                pl.BlockSpec((NB, 1, H), lambda i: (0, 0, 0)),
                pl.BlockSpec((2, H), lambda i: (0, 0)),
            ],
            out_specs=pl.BlockSpec((G, bn, 128), lambda i: (0, i, 0)),
            out_shape=jax.ShapeDtypeStruct((G, N, 128), jnp.float32),
        )(z, h, psum, psumsq, gb)

    def body(z_ref, h_ref, ps_ref, pq_ref, gb_ref, out_ref):
        mu, inv = stats(ps_ref, pq_ref)
        zn = gb_ref[0] * (z_ref[...] - mu) * inv + gb_ref[1]
        r = jnp.maximum(zn, 0.0)
        hfull = jnp.concatenate([h_ref[g] for g in range(G)], axis=1)
        out_ref[...] = hfull + r

    return pl.pallas_call(
        body,
        grid=(NB,),
        in_specs=[
            pl.BlockSpec((bn, H), lambda i: (i, 0)),
            pl.BlockSpec((G, bn, 128), lambda i: (0, i, 0)),
            pl.BlockSpec((NB, 1, H), lambda i: (0, 0, 0)),
            pl.BlockSpec((NB, 1, H), lambda i: (0, 0, 0)),
            pl.BlockSpec((2, H), lambda i: (0, 0)),
        ],
        out_specs=pl.BlockSpec((bn, H), lambda i: (i, 0)),
        out_shape=jax.ShapeDtypeStruct((N, H), jnp.float32),
    )(z, h, psum, psumsq, gb)


def kernel(x, edge_index, W_rel0, b_rel0, W_root0, gamma0, beta0,
           W_rel, b_rel, W_root, gamma, beta):
    N, D = x.shape
    H = W_rel0.shape[1]
    G0 = D // 128
    G = H // 128
    E = edge_index.shape[1]
    bn = 1000

    src = edge_index[0].astype(jnp.int32)
    dst = edge_index[1].astype(jnp.int32)
    e_pad = ((E + _K - 1) // _K) * _K
    if e_pad != E:
        pad_pos = jnp.arange(E, e_pad, dtype=jnp.int32)
        src = jnp.concatenate([src, jnp.zeros((e_pad - E,), jnp.int32)])
        dst = jnp.concatenate([dst, N + pad_pos % 16])

    x_b = x.reshape(N, G0, 128).transpose(1, 0, 2)

    agg = _segment_sum_cols(x_b.reshape(G0 * N, 128), src, dst, G0, N)
    zr = _tc_root(x_b, W_root0.reshape(G0, 128, H), b_rel0, bn)
    z, ps, pq = _tc_rel(agg.reshape(G0, N, 128), zr,
                        W_rel0.reshape(G0, 128, H), bn)
    out = _tc_bn(z, None, ps, pq, gamma0, beta0, bn, "first")

    for i in range(4):
        agg = _segment_sum_cols(out.reshape(G * N, 128), src, dst, G, N)
        zr = _tc_root(out, W_root[i].reshape(G, 128, H), b_rel[i], bn)
        z, ps, pq = _tc_rel(agg.reshape(G, N, 128), zr,
                            W_rel[i].reshape(G, 128, H), bn)
        mode = "last" if i == 3 else "mid"
        out = _tc_bn(z, out, ps, pq, gamma[i], beta[i], bn, mode)

    return out
